# Initial kernel scaffold; baseline (speedup 1.0000x reference)
import jax
import jax.numpy as jnp
from jax.experimental import pallas as pl


def kernel(token_ids, w):
    raise NotImplementedError("write your pallas kernel here")


# speedup vs baseline: 1.1022x; 1.1022x over previous
import functools

import jax
import jax.numpy as jnp
from jax import lax
from jax.experimental import pallas as pl
from jax.experimental.pallas import tpu as pltpu
from jax.experimental.pallas import tpu_sc as plsc

VOCAB = 1000000
D = 32
B_TOTAL = 16384 * 50

_info = plsc.get_sparse_core_info()
_NC, _NS = _info.num_cores, _info.num_subcores
_NW = _NC * _NS
_B_PER_W = B_TOTAL // _NW
_C = 1024
_N_CHUNKS = _B_PER_W // _C

_mesh = plsc.VectorSubcoreMesh(core_axis_name="c", subcore_axis_name="s")


@functools.partial(
    pl.kernel,
    mesh=_mesh,
    out_type=jax.ShapeDtypeStruct((B_TOTAL, D), jnp.float32),
    scratch_types=[
        pltpu.VMEM((_B_PER_W,), jnp.int32),
        pltpu.VMEM((_C, D), jnp.float32),
        pltpu.SemaphoreType.DMA,
    ],
    compiler_params=pltpu.CompilerParams(use_tc_tiling_on_sc=False),
)
def _embed_lookup(idx_hbm, table_hbm, out_hbm, idx_v, buf, sem):
    wid = lax.axis_index("s") * _NC + lax.axis_index("c")
    base = wid * _B_PER_W
    pltpu.sync_copy(idx_hbm.at[pl.ds(base, _B_PER_W)], idx_v)

    def step(c, carry):
        off = c * _C
        pltpu.async_copy(
            table_hbm.at[idx_v.at[pl.ds(off, _C)]], buf, sem
        ).wait()
        pltpu.sync_copy(buf, out_hbm.at[pl.ds(base + off, _C)])
        return carry

    lax.fori_loop(0, _N_CHUNKS, step, 0)


def kernel(token_ids, w):
    idx = token_ids.reshape(-1).astype(jnp.int32)
    out = _embed_lookup(idx, w)
    return out.reshape((*token_ids.shape, D))


# speedup vs baseline: 1.3813x; 1.2532x over previous
import functools

import jax
import jax.numpy as jnp
from jax import lax
from jax.experimental import pallas as pl
from jax.experimental.pallas import tpu as pltpu
from jax.experimental.pallas import tpu_sc as plsc

VOCAB = 1000000
D = 32
BATCH = 16384
HIST = 50

_info = plsc.get_sparse_core_info()
_NC, _NS = _info.num_cores, _info.num_subcores
_NW = _NC * _NS
_NB = BATCH // _NW
_IDX_PER_W = _NB * HIST

_mesh = plsc.VectorSubcoreMesh(core_axis_name="c", subcore_axis_name="s")


@functools.partial(
    pl.kernel,
    mesh=_mesh,
    out_type=jax.ShapeDtypeStruct((HIST, D, BATCH), jnp.float32),
    scratch_types=[
        pltpu.VMEM((_IDX_PER_W,), jnp.int32),
        pltpu.VMEM((_NB,), jnp.int32),
        pltpu.VMEM((_NB, D), jnp.float32),
        pltpu.VMEM((D, _NB), jnp.float32),
        pltpu.SemaphoreType.DMA,
    ],
    compiler_params=pltpu.CompilerParams(
        use_tc_tiling_on_sc=False, needs_layout_passes=False),
)
def _embed_lookup(idx_hbm, table_hbm, out_hbm, idx_v, col_v, gath_v, tr_v, sem):
    wid = lax.axis_index("s") * _NC + lax.axis_index("c")
    b0 = wid * _NB
    pltpu.sync_copy(idx_hbm.at[pl.ds(wid * _IDX_PER_W, _IDX_PER_W)], idx_v)
    iota = lax.iota(jnp.int32, 16)

    def do_h(h, carry):
        def ext(jb, c):
            rows = (jb * 16 + iota) * HIST + h
            col_v[pl.ds(jb * 16, 16)] = plsc.load_gather(idx_v, [rows])
            return c

        lax.fori_loop(0, _NB // 16, ext, 0)
        pltpu.async_copy(table_hbm.at[col_v], gath_v, sem).wait()

        def trd(d, c):
            cols = jnp.full((16,), d, jnp.int32)
            for jb in range(_NB // 16):
                rows = jb * 16 + iota
                tr_v[d, pl.ds(jb * 16, 16)] = plsc.load_gather(
                    gath_v, [rows, cols])
            return c

        lax.fori_loop(0, D, trd, 0)
        pltpu.sync_copy(tr_v, out_hbm.at[h, :, pl.ds(b0, _NB)])
        return carry

    lax.fori_loop(0, HIST, do_h, 0)


def kernel(token_ids, w):
    idx = token_ids.reshape(-1).astype(jnp.int32)
    out_t = _embed_lookup(idx, w)
    return jnp.transpose(out_t, (2, 0, 1))
